# baseline (speedup 1.0000x reference)
import jax
import jax.numpy as jnp
from jax import lax
from jax.experimental import pallas as pl
from jax.experimental.pallas import tpu as pltpu
from jax.experimental.pallas import tpu_sc as plsc

_LANES = 16
_UNROLL = 16
_NOB = 4


def kernel(indices, E):
    B, H = indices.shape
    V, D = E.shape
    E_T = E.T
    idx_T = indices.T

    mesh = plsc.VectorSubcoreMesh(core_axis_name="core",
                                  subcore_axis_name="subcore")
    n_sub = 32
    d_per = D // n_sub
    assert (H - 2) % _NOB == 0 and H >= _NOB + 2
    n_grp = (H - 2) // _NOB

    @pl.kernel(
        out_type=jax.ShapeDtypeStruct((H, D, B), E.dtype),
        mesh=mesh,
        scratch_types=[
            pltpu.VMEM((V,), E.dtype),
            pltpu.VMEM((B,), indices.dtype),
            pltpu.VMEM((B,), indices.dtype),
        ] + [pltpu.VMEM((B,), E.dtype)] * _NOB + [
            pltpu.SemaphoreType.DMA,
            pltpu.SemaphoreType.DMA,
            pltpu.SemaphoreType.DMA,
        ] + [pltpu.SemaphoreType.DMA] * _NOB,
        compiler_params=pltpu.CompilerParams(use_tc_tiling_on_sc=False,
                                             needs_layout_passes=False),
    )
    def gather_kernel(et_hbm, it_hbm, o_hbm, row, ib0, ib1, *rest):
        obs = rest[:_NOB]
        sem_row, sem_i0, sem_i1 = rest[_NOB:_NOB + 3]
        sem_os = rest[_NOB + 3:]
        c = lax.axis_index("core")
        s = lax.axis_index("subcore")
        t = c * 16 + s

        def gather_into(ob, ib):
            @plsc.parallel_loop(0, B, step=_LANES, unroll=_UNROLL)
            def _(i):
                sl = pl.ds(i, _LANES)
                ob[sl] = plsc.load_gather(row, [ib[sl]])

        @pl.loop(0, d_per)
        def _(j):
            d = t * d_per + j
            pltpu.make_async_copy(et_hbm.at[d], row, sem_row).start()
            pltpu.make_async_copy(it_hbm.at[0], ib0, sem_i0).start()
            pltpu.make_async_copy(it_hbm.at[1], ib1, sem_i1).start()
            pltpu.make_async_copy(et_hbm.at[d], row, sem_row).wait()

            @pl.loop(0, n_grp)
            def _(g):
                base = g * _NOB
                for u in range(_NOB):
                    h = base + u
                    ib, sem_i = (ib0, sem_i0) if u % 2 == 0 else (ib1, sem_i1)
                    ob, sem_o = obs[u], sem_os[u]
                    pltpu.make_async_copy(it_hbm.at[h], ib, sem_i).wait()

                    @pl.when(g > 0)
                    def _():
                        pltpu.make_async_copy(ob, o_hbm.at[h - _NOB, d],
                                              sem_o).wait()

                    gather_into(ob, ib)
                    pltpu.make_async_copy(ob, o_hbm.at[h, d], sem_o).start()

                    @pl.when(h + 2 < H)
                    def _():
                        pltpu.make_async_copy(it_hbm.at[h + 2], ib,
                                              sem_i).start()

            for u in range(2):
                h = H - 2 + u
                ib, sem_i = (ib0, sem_i0) if u % 2 == 0 else (ib1, sem_i1)
                ob, sem_o = obs[u], sem_os[u]
                pltpu.make_async_copy(it_hbm.at[h], ib, sem_i).wait()
                pltpu.make_async_copy(ob, o_hbm.at[h - _NOB, d], sem_o).wait()
                gather_into(ob, ib)
                pltpu.make_async_copy(ob, o_hbm.at[h, d], sem_o).start()

            for u in range(2):
                pltpu.make_async_copy(obs[u], o_hbm.at[H - 2 + u, d],
                                      sem_os[u]).wait()
            for u in range(2, _NOB):
                pltpu.make_async_copy(obs[u], o_hbm.at[H - 2 - _NOB + u, d],
                                      sem_os[u]).wait()

    out = gather_kernel(E_T, idx_T)
    return jnp.transpose(out, (2, 0, 1))
